# baseline (speedup 1.0000x reference)
import dataclasses

import jax
import jax.numpy as jnp
from jax import lax
from jax.experimental import pallas as pl
from jax.experimental.pallas import tpu as pltpu
from jax.experimental.pallas import tpu_sc as plsc

N = 320000
D = 128
S = 10000
L = 16
NW = 32
SEG_PER = 320
S_PAD = NW * SEG_PER
C = 256
HC = 128
NB = 48
SC_SEGS = 16 * SEG_PER
ACC_ROWS = SC_SEGS + 128
ZROWS = ACC_ROWS // 16
DUMP = SC_SEGS
EBLOCKS = ((0, 128), (128, 128), (256, 64))
ZBLOCKS = ((0, 128), (128, 128), (256, 72))


def _sc_body(x_hbm, bounds_hbm, ids_hbm, out_hbm,
             xbuf0, xbuf1, idb0, idb1, ix0, ix1, cnt_ref, bounds_vmem,
             acc_sh, sem0, sem1, ss0, ss1):
    core = lax.axis_index("c")
    sub = lax.axis_index("s")
    w = core * 16 + sub
    s_lo = w * SEG_PER
    sc_base = core * SC_SEGS
    iota = lax.iota(jnp.int32, L)
    ones = jnp.ones((L,), jnp.float32)

    pltpu.sync_copy(bounds_hbm, bounds_vmem.at[pl.ds(0, NB)])
    bv = bounds_vmem[pl.ds(w, L)]
    r_lo = bv[0]
    r_hi = bv[1]

    @pl.loop(0, SEG_PER, step=L)
    def _(k):
        cnt_ref[pl.ds(k, L)] = jnp.zeros((L,), jnp.float32)

    @pl.loop(0, HC)
    def _(r):
        for j in range(D // L):
            xbuf0[r, pl.ds(j * L, L)] = jnp.zeros((L,), jnp.float32)

    for off, sz in ZBLOCKS:
        pltpu.sync_copy(xbuf0.at[pl.ds(0, sz)],
                        acc_sh.at[pl.ds(sub * ZROWS + off, sz)])
    plsc.subcore_barrier()

    base = (r_lo // 16) * 16
    nchunks = (r_hi - base + C - 1) // C

    def chunk_start(cix):
        return jnp.minimum(base + cix * C, N - C)

    def issue(cix, xb, ib, sem):
        start = chunk_start(cix)
        pltpu.async_copy(x_hbm.at[pl.ds(start, C)], xb, sem)
        pltpu.async_copy(ids_hbm.at[pl.ds(start, C)], ib.at[pl.ds(0, C)], sem)

    def wait(xb, ib, sem):
        start = chunk_start(0)
        pltpu.make_async_copy(x_hbm.at[pl.ds(start, C)], xb, sem).wait()
        pltpu.make_async_copy(
            ids_hbm.at[pl.ds(start, C)], ib.at[pl.ds(0, C)], sem).wait()

    def build_ix(cix, ib, ix):
        start = chunk_start(cix)
        lo = jnp.maximum(r_lo, base + cix * C)
        hi = jnp.minimum(base + (cix + 1) * C, r_hi)
        for h in range(C // HC):
            for o in range(0, HC, L):
                iv = ib[pl.ds(h * HC + o, L)]
                rowv = start + h * HC + o + iota
                m = (rowv >= lo) & (rowv < hi)
                ix[h, pl.ds(o, L)] = jnp.where(m, iv - sc_base, DUMP)
                plsc.addupdate_scatter(cnt_ref, [iv - s_lo], ones, mask=m)

    def issue_scatter(xb, ix, ssem):
        for h in range(C // HC):
            pltpu.async_copy(xb.at[pl.ds(h * HC, HC)],
                             acc_sh.at[ix.at[h]], ssem, add=True)

    def drain_scatter(xb, ix, ssem):
        for h in range(C // HC):
            pltpu.make_async_copy(xb.at[pl.ds(h * HC, HC)],
                                  acc_sh.at[ix.at[h]], ssem).wait()

    @pl.when(nchunks > 0)
    def _():
        issue(0, xbuf0, idb0, sem0)

    def pair_body(p, _):
        c0 = 2 * p
        c1 = c0 + 1
        wait(xbuf0, idb0, sem0)

        @pl.when(c1 < nchunks)
        def _():
            @pl.when(p >= 1)
            def _():
                drain_scatter(xbuf1, ix1, ss1)

            issue(c1, xbuf1, idb1, sem1)

        build_ix(c0, idb0, ix0)
        issue_scatter(xbuf0, ix0, ss0)

        @pl.when(c1 < nchunks)
        def _():
            wait(xbuf1, idb1, sem1)
            build_ix(c1, idb1, ix1)
            drain_scatter(xbuf0, ix0, ss0)

            @pl.when(c1 + 1 < nchunks)
            def _():
                issue(c1 + 1, xbuf0, idb0, sem0)

            issue_scatter(xbuf1, ix1, ss1)

        @pl.when(c1 >= nchunks)
        def _():
            drain_scatter(xbuf0, ix0, ss0)

        return 0

    lax.fori_loop(0, (nchunks + 1) // 2, pair_body, 0)

    @pl.when(nchunks >= 2)
    def _():
        drain_scatter(xbuf1, ix1, ss1)

    plsc.subcore_barrier()

    for off, sz in EBLOCKS:
        pltpu.sync_copy(acc_sh.at[pl.ds(sub * SEG_PER + off, sz)],
                        xbuf0.at[pl.ds(0, sz)])

        @pl.loop(0, sz)
        def _(k):
            cv = plsc.load_gather(cnt_ref, [jnp.full((L,), off + k)])
            inv = 1.0 / jnp.maximum(cv, 1.0)
            vs = [xbuf0[k, pl.ds(j * L, L)] for j in range(D // L)]
            for j, v in enumerate(vs):
                xbuf0[k, pl.ds(j * L, L)] = v * inv

        pltpu.sync_copy(xbuf0.at[pl.ds(0, sz)],
                        out_hbm.at[pl.ds(s_lo + off, sz)])


def _make_sc_kernel(interpret=False):
    mesh = plsc.VectorSubcoreMesh(
        core_axis_name="c", subcore_axis_name="s", num_cores=2, num_subcores=16
    )
    cp = pltpu.CompilerParams()
    if "needs_layout_passes" in pltpu.CompilerParams.__dataclass_fields__:
        cp = dataclasses.replace(cp, needs_layout_passes=False)
    return pl.kernel(
        _sc_body,
        out_type=jax.ShapeDtypeStruct((S_PAD, D), jnp.float32),
        mesh=mesh,
        scratch_types=[
            pltpu.VMEM((C, D), jnp.float32),
            pltpu.VMEM((C, D), jnp.float32),
            pltpu.VMEM((C + L,), jnp.int32),
            pltpu.VMEM((C + L,), jnp.int32),
            pltpu.VMEM((C // HC, HC), jnp.int32),
            pltpu.VMEM((C // HC, HC), jnp.int32),
            pltpu.VMEM((SEG_PER,), jnp.float32),
            pltpu.VMEM((NB + L,), jnp.int32),
            pltpu.VMEM_SHARED((ACC_ROWS, D), jnp.float32),
            pltpu.SemaphoreType.DMA,
            pltpu.SemaphoreType.DMA,
            pltpu.SemaphoreType.DMA,
            pltpu.SemaphoreType.DMA,
        ],
        compiler_params=cp,
        interpret=interpret,
    )


@jax.jit
def kernel(X, cluster_assignment):
    ids = cluster_assignment.astype(jnp.int32)
    queries = jnp.arange(NB, dtype=jnp.int32) * SEG_PER
    bounds = jnp.searchsorted(ids, queries, side="left").astype(jnp.int32)
    out = _make_sc_kernel()(X, bounds, ids)
    return out[:S]


# speedup vs baseline: 1.2956x; 1.2956x over previous
import dataclasses

import jax
import jax.numpy as jnp
from jax import lax
from jax.experimental import pallas as pl
from jax.experimental.pallas import tpu as pltpu
from jax.experimental.pallas import tpu_sc as plsc

N = 320000
D = 128
S = 10000
L = 16
NW = 32
SEG_PER = 320
S_PAD = NW * SEG_PER
C = 256
HC = 128
NB = 48
SC_SEGS = 16 * SEG_PER
ACC_ROWS = SC_SEGS + 128
ZROWS = ACC_ROWS // 16
DUMP = SC_SEGS
EBLOCKS = ((0, 128), (128, 128), (256, 64))
ZBLOCKS = ((0, 128), (128, 128), (256, 72))


def _sc_body(x_hbm, bounds_hbm, ids_hbm, out_hbm,
             xbuf0, xbuf1, idb0, idb1, ix0, ix1, cnt_ref, bounds_vmem,
             acc_sh, sem0, sem1):
    core = lax.axis_index("c")
    sub = lax.axis_index("s")
    w = core * 16 + sub
    s_lo = w * SEG_PER
    sc_base = core * SC_SEGS
    iota = lax.iota(jnp.int32, L)
    ones = jnp.ones((L,), jnp.float32)

    pltpu.sync_copy(bounds_hbm, bounds_vmem.at[pl.ds(0, NB)])
    bv = bounds_vmem[pl.ds(w, L)]
    r_lo = bv[0]
    r_hi = bv[1]

    @pl.loop(0, SEG_PER, step=L)
    def _(k):
        cnt_ref[pl.ds(k, L)] = jnp.zeros((L,), jnp.float32)

    @pl.loop(0, HC)
    def _(r):
        for j in range(D // L):
            xbuf0[r, pl.ds(j * L, L)] = jnp.zeros((L,), jnp.float32)

    for off, sz in ZBLOCKS:
        pltpu.sync_copy(xbuf0.at[pl.ds(0, sz)],
                        acc_sh.at[pl.ds(sub * ZROWS + off, sz)])
    plsc.subcore_barrier()

    base = (r_lo // 16) * 16
    nchunks = (r_hi - base + C - 1) // C

    def chunk_start(cix):
        return jnp.minimum(base + cix * C, N - C)

    def issue(cix, xb, ib, sem):
        start = chunk_start(cix)
        pltpu.async_copy(x_hbm.at[pl.ds(start, C)], xb, sem)
        pltpu.async_copy(ids_hbm.at[pl.ds(start, C)], ib.at[pl.ds(0, C)], sem)

    def wait(xb, ib, sem):
        start = chunk_start(0)
        pltpu.make_async_copy(x_hbm.at[pl.ds(start, C)], xb, sem).wait()
        pltpu.make_async_copy(
            ids_hbm.at[pl.ds(start, C)], ib.at[pl.ds(0, C)], sem).wait()

    def process(cix, xb, ib, ix):
        start = chunk_start(cix)
        lo = jnp.maximum(r_lo, base + cix * C)
        hi = jnp.minimum(base + (cix + 1) * C, r_hi)

        for h in range(C // HC):
            for o in range(0, HC, L):
                iv = ib[pl.ds(h * HC + o, L)]
                rowv = start + h * HC + o + iota
                m = (rowv >= lo) & (rowv < hi)
                ix[h, pl.ds(o, L)] = jnp.where(m, iv - sc_base, DUMP)
                plsc.addupdate_scatter(cnt_ref, [iv - s_lo], ones, mask=m)

        for h in range(C // HC):
            pltpu.sync_copy(xb.at[pl.ds(h * HC, HC)],
                            acc_sh.at[ix.at[h]], add=True)

    @pl.when(nchunks > 0)
    def _():
        issue(0, xbuf0, idb0, sem0)

    def pair_body(p, _):
        c0 = 2 * p
        c1 = c0 + 1
        wait(xbuf0, idb0, sem0)

        @pl.when(c1 < nchunks)
        def _():
            issue(c1, xbuf1, idb1, sem1)

        process(c0, xbuf0, idb0, ix0)

        @pl.when(c1 < nchunks)
        def _():
            wait(xbuf1, idb1, sem1)

            @pl.when(c1 + 1 < nchunks)
            def _():
                issue(c1 + 1, xbuf0, idb0, sem0)

            process(c1, xbuf1, idb1, ix1)

        return 0

    lax.fori_loop(0, (nchunks + 1) // 2, pair_body, 0)

    plsc.subcore_barrier()

    for off, sz in EBLOCKS:
        pltpu.sync_copy(acc_sh.at[pl.ds(sub * SEG_PER + off, sz)],
                        xbuf0.at[pl.ds(0, sz)])

        @pl.loop(0, sz)
        def _(k):
            cv = plsc.load_gather(cnt_ref, [jnp.full((L,), off + k)])
            inv = 1.0 / jnp.maximum(cv, 1.0)
            vs = [xbuf0[k, pl.ds(j * L, L)] for j in range(D // L)]
            for j, v in enumerate(vs):
                xbuf0[k, pl.ds(j * L, L)] = v * inv

        pltpu.sync_copy(xbuf0.at[pl.ds(0, sz)],
                        out_hbm.at[pl.ds(s_lo + off, sz)])


def _make_sc_kernel(interpret=False):
    mesh = plsc.VectorSubcoreMesh(
        core_axis_name="c", subcore_axis_name="s", num_cores=2, num_subcores=16
    )
    cp = pltpu.CompilerParams()
    if "needs_layout_passes" in pltpu.CompilerParams.__dataclass_fields__:
        cp = dataclasses.replace(cp, needs_layout_passes=False)
    return pl.kernel(
        _sc_body,
        out_type=jax.ShapeDtypeStruct((S_PAD, D), jnp.float32),
        mesh=mesh,
        scratch_types=[
            pltpu.VMEM((C, D), jnp.float32),
            pltpu.VMEM((C, D), jnp.float32),
            pltpu.VMEM((C + L,), jnp.int32),
            pltpu.VMEM((C + L,), jnp.int32),
            pltpu.VMEM((C // HC, HC), jnp.int32),
            pltpu.VMEM((C // HC, HC), jnp.int32),
            pltpu.VMEM((SEG_PER,), jnp.float32),
            pltpu.VMEM((NB + L,), jnp.int32),
            pltpu.VMEM_SHARED((ACC_ROWS, D), jnp.float32),
            pltpu.SemaphoreType.DMA,
            pltpu.SemaphoreType.DMA,
        ],
        compiler_params=cp,
        interpret=interpret,
    )


@jax.jit
def kernel(X, cluster_assignment):
    ids = cluster_assignment.astype(jnp.int32)
    queries = jnp.arange(NB, dtype=jnp.int32) * SEG_PER
    bounds = jnp.searchsorted(ids, queries, side="left",
                              method="compare_all").astype(jnp.int32)
    out = _make_sc_kernel()(X, bounds, ids)
    return out[:S]


# speedup vs baseline: 1.3052x; 1.0074x over previous
import dataclasses

import jax
import jax.numpy as jnp
from jax import lax
from jax.experimental import pallas as pl
from jax.experimental.pallas import tpu as pltpu
from jax.experimental.pallas import tpu_sc as plsc

N = 320000
D = 128
S = 10000
L = 16
NW = 32
SEG_PER = 320
S_PAD = NW * SEG_PER
C = 256
HC = 128
NB = 48
SC_SEGS = 16 * SEG_PER
ACC_ROWS = SC_SEGS + 128
ZROWS = ACC_ROWS // 16
DUMP = SC_SEGS
EBLOCKS = ((0, 128), (128, 128), (256, 64))
ZBLOCKS = ((0, 128), (128, 128), (256, 72))


def _sc_body(x_hbm, bounds_hbm, ids_hbm, out_hbm,
             xbuf0, xbuf1, idb0, idb1, ix0, ix1, cnt_ref, bounds_vmem,
             acc_sh, sem0, sem1):
    core = lax.axis_index("c")
    sub = lax.axis_index("s")
    w = core * 16 + sub
    s_lo = w * SEG_PER
    sc_base = core * SC_SEGS
    iota = lax.iota(jnp.int32, L)
    ones = jnp.ones((L,), jnp.float32)

    pltpu.sync_copy(bounds_hbm, bounds_vmem.at[pl.ds(0, NB)])
    bv = bounds_vmem[pl.ds(w, L)]
    r_lo = bv[0]
    r_hi = bv[1]

    @pl.loop(0, SEG_PER, step=L)
    def _(k):
        cnt_ref[pl.ds(k, L)] = jnp.zeros((L,), jnp.float32)

    @pl.loop(0, HC)
    def _(r):
        for j in range(D // L):
            xbuf0[r, pl.ds(j * L, L)] = jnp.zeros((L,), jnp.float32)

    for off, sz in ZBLOCKS:
        pltpu.sync_copy(xbuf0.at[pl.ds(0, sz)],
                        acc_sh.at[pl.ds(sub * ZROWS + off, sz)])
    plsc.subcore_barrier()

    base = (r_lo // 16) * 16
    nchunks = (r_hi - base + C - 1) // C

    def chunk_start(cix):
        return jnp.minimum(base + cix * C, N - C)

    def issue(cix, xb, ib, sem):
        start = chunk_start(cix)
        pltpu.async_copy(x_hbm.at[pl.ds(start, C)], xb, sem)
        pltpu.async_copy(ids_hbm.at[pl.ds(start, C)], ib.at[pl.ds(0, C)], sem)

    def wait(xb, ib, sem):
        start = chunk_start(0)
        pltpu.make_async_copy(x_hbm.at[pl.ds(start, C)], xb, sem).wait()
        pltpu.make_async_copy(
            ids_hbm.at[pl.ds(start, C)], ib.at[pl.ds(0, C)], sem).wait()

    def process(cix, xb, ib, ix):
        start = chunk_start(cix)
        lo = jnp.maximum(r_lo, base + cix * C)
        hi = jnp.minimum(base + (cix + 1) * C, r_hi)

        for h in range(C // HC):
            for o in range(0, HC, L):
                iv = ib[pl.ds(h * HC + o, L)]
                rowv = start + h * HC + o + iota
                m = (rowv >= lo) & (rowv < hi)
                ix[h, pl.ds(o, L)] = jnp.where(m, iv - sc_base, DUMP)
                plsc.addupdate_scatter(cnt_ref, [iv - s_lo], ones, mask=m)

        for h in range(C // HC):
            pltpu.sync_copy(xb.at[pl.ds(h * HC, HC)],
                            acc_sh.at[ix.at[h]], add=True)

    @pl.when(nchunks > 0)
    def _():
        issue(0, xbuf0, idb0, sem0)

    def pair_body(p, _):
        c0 = 2 * p
        c1 = c0 + 1
        wait(xbuf0, idb0, sem0)

        @pl.when(c1 < nchunks)
        def _():
            issue(c1, xbuf1, idb1, sem1)

        process(c0, xbuf0, idb0, ix0)

        @pl.when(c1 < nchunks)
        def _():
            wait(xbuf1, idb1, sem1)

            @pl.when(c1 + 1 < nchunks)
            def _():
                issue(c1 + 1, xbuf0, idb0, sem0)

            process(c1, xbuf1, idb1, ix1)

        return 0

    lax.fori_loop(0, (nchunks + 1) // 2, pair_body, 0)

    plsc.subcore_barrier()

    for off, sz in EBLOCKS:
        pltpu.sync_copy(acc_sh.at[pl.ds(sub * SEG_PER + off, sz)],
                        xbuf0.at[pl.ds(0, sz)])

        @pl.loop(0, sz)
        def _(k):
            cv = plsc.load_gather(cnt_ref, [jnp.full((L,), off + k)])
            inv = 1.0 / jnp.maximum(cv, 1.0)
            vs = [xbuf0[k, pl.ds(j * L, L)] for j in range(D // L)]
            for j, v in enumerate(vs):
                xbuf0[k, pl.ds(j * L, L)] = v * inv

        pltpu.sync_copy(xbuf0.at[pl.ds(0, sz)],
                        out_hbm.at[pl.ds(s_lo + off, sz)])


def _make_sc_kernel(interpret=False):
    mesh = plsc.VectorSubcoreMesh(
        core_axis_name="c", subcore_axis_name="s", num_cores=2, num_subcores=16
    )
    cp = pltpu.CompilerParams()
    if "needs_layout_passes" in pltpu.CompilerParams.__dataclass_fields__:
        cp = dataclasses.replace(cp, needs_layout_passes=False)
    return pl.kernel(
        _sc_body,
        out_type=jax.ShapeDtypeStruct((S_PAD, D), jnp.float32),
        mesh=mesh,
        scratch_types=[
            pltpu.VMEM((C, D), jnp.float32),
            pltpu.VMEM((C, D), jnp.float32),
            pltpu.VMEM((C + L,), jnp.int32),
            pltpu.VMEM((C + L,), jnp.int32),
            pltpu.VMEM((C // HC, HC), jnp.int32),
            pltpu.VMEM((C // HC, HC), jnp.int32),
            pltpu.VMEM((SEG_PER,), jnp.float32),
            pltpu.VMEM((NB + L,), jnp.int32),
            pltpu.VMEM_SHARED((ACC_ROWS, D), jnp.float32),
            pltpu.SemaphoreType.DMA,
            pltpu.SemaphoreType.DMA,
        ],
        compiler_params=cp,
        interpret=interpret,
    )


@jax.jit
def kernel(X, cluster_assignment):
    ids = cluster_assignment.astype(jnp.int32)
    queries = jnp.arange(NW + 1, dtype=jnp.int32) * SEG_PER
    bounds = jnp.searchsorted(ids, queries, side="left",
                              method="compare_all").astype(jnp.int32)
    bounds = jnp.concatenate(
        [bounds, jnp.zeros((NB - NW - 1,), jnp.int32)])
    out = _make_sc_kernel()(X, bounds, ids)
    return out[:S]


# speedup vs baseline: 1.3359x; 1.0235x over previous
import dataclasses

import jax
import jax.numpy as jnp
from jax import lax
from jax.experimental import pallas as pl
from jax.experimental.pallas import tpu as pltpu
from jax.experimental.pallas import tpu_sc as plsc

N = 320000
D = 128
S = 10000
L = 16
NW = 32
SEG_PER = 320
S_PAD = NW * SEG_PER
C = 256
HC = 128
NB = 48
SC_SEGS = 16 * SEG_PER
ACC_ROWS = SC_SEGS + 128
ZROWS = ACC_ROWS // 16
DUMP = SC_SEGS
EBLOCKS = ((0, 128), (128, 128), (256, 64))
ZBLOCKS = ((0, 128), (128, 128), (256, 72))


def _sc_body(x_hbm, bounds_hbm, ids_hbm, out_hbm,
             xbuf0, xbuf1, idb0, idb1, ix0, ix1, cnt_ref, bounds_vmem,
             acc_sh, sem0, sem1):
    core = lax.axis_index("c")
    sub = lax.axis_index("s")
    w = core * 16 + sub
    s_lo = w * SEG_PER
    sc_base = core * SC_SEGS
    iota = lax.iota(jnp.int32, L)
    ones = jnp.ones((L,), jnp.float32)

    pltpu.sync_copy(bounds_hbm, bounds_vmem.at[pl.ds(0, NB)])
    bv = bounds_vmem[pl.ds(w, L)]
    r_lo = bv[0]
    r_hi = bv[1]

    @pl.loop(0, SEG_PER, step=L)
    def _(k):
        cnt_ref[pl.ds(k, L)] = jnp.zeros((L,), jnp.float32)

    @pl.loop(0, HC)
    def _(r):
        for j in range(D // L):
            xbuf0[r, pl.ds(j * L, L)] = jnp.zeros((L,), jnp.float32)

    for off, sz in ZBLOCKS:
        pltpu.sync_copy(xbuf0.at[pl.ds(0, sz)],
                        acc_sh.at[pl.ds(sub * ZROWS + off, sz)])
    plsc.subcore_barrier()

    base = (r_lo // 16) * 16
    nchunks = (r_hi - base + C - 1) // C

    def chunk_start(cix):
        return jnp.minimum(base + cix * C, N - C)

    def issue(cix, xb, ib, sem):
        start = chunk_start(cix)
        pltpu.async_copy(x_hbm.at[pl.ds(start, C)], xb, sem)
        pltpu.async_copy(ids_hbm.at[pl.ds(start, C)], ib.at[pl.ds(0, C)], sem)

    def wait(xb, ib, sem):
        start = chunk_start(0)
        pltpu.make_async_copy(x_hbm.at[pl.ds(start, C)], xb, sem).wait()
        pltpu.make_async_copy(
            ids_hbm.at[pl.ds(start, C)], ib.at[pl.ds(0, C)], sem).wait()

    def process(cix, xb, ib, ix):
        start = chunk_start(cix)
        lo = jnp.maximum(r_lo, base + cix * C)
        hi = jnp.minimum(base + (cix + 1) * C, r_hi)

        for h in range(C // HC):
            for o in range(0, HC, L):
                iv = ib[pl.ds(h * HC + o, L)]
                rowv = start + h * HC + o + iota
                m = (rowv >= lo) & (rowv < hi)
                ix[h, pl.ds(o, L)] = jnp.where(m, iv - sc_base, DUMP)
                plsc.addupdate_scatter(cnt_ref, [iv - s_lo], ones, mask=m)

        for h in range(C // HC):
            pltpu.sync_copy(xb.at[pl.ds(h * HC, HC)],
                            acc_sh.at[ix.at[h]], add=True)

    @pl.when(nchunks > 0)
    def _():
        issue(0, xbuf0, idb0, sem0)

    def pair_body(p, _):
        c0 = 2 * p
        c1 = c0 + 1
        wait(xbuf0, idb0, sem0)

        @pl.when(c1 < nchunks)
        def _():
            issue(c1, xbuf1, idb1, sem1)

        process(c0, xbuf0, idb0, ix0)

        @pl.when(c1 < nchunks)
        def _():
            wait(xbuf1, idb1, sem1)

            @pl.when(c1 + 1 < nchunks)
            def _():
                issue(c1 + 1, xbuf0, idb0, sem0)

            process(c1, xbuf1, idb1, ix1)

        return 0

    lax.fori_loop(0, (nchunks + 1) // 2, pair_body, 0)

    plsc.subcore_barrier()

    for off, sz in EBLOCKS:
        pltpu.sync_copy(acc_sh.at[pl.ds(sub * SEG_PER + off, sz)],
                        xbuf0.at[pl.ds(0, sz)])

        @pl.loop(0, sz)
        def _(k):
            cv = plsc.load_gather(cnt_ref, [jnp.full((L,), off + k)])
            inv = 1.0 / jnp.maximum(cv, 1.0)
            vs = [xbuf0[k, pl.ds(j * L, L)] for j in range(D // L)]
            for j, v in enumerate(vs):
                xbuf0[k, pl.ds(j * L, L)] = v * inv

        @pl.when(w < NW - 1)
        def _():
            pltpu.sync_copy(xbuf0.at[pl.ds(0, sz)],
                            out_hbm.at[pl.ds(s_lo + off, sz)])

        if off == 0:
            @pl.when(w == NW - 1)
            def _():
                pltpu.sync_copy(xbuf0.at[pl.ds(0, S - (NW - 1) * SEG_PER)],
                                out_hbm.at[pl.ds(s_lo, S - (NW - 1) * SEG_PER)])


def _make_sc_kernel(interpret=False):
    mesh = plsc.VectorSubcoreMesh(
        core_axis_name="c", subcore_axis_name="s", num_cores=2, num_subcores=16
    )
    cp = pltpu.CompilerParams()
    if "needs_layout_passes" in pltpu.CompilerParams.__dataclass_fields__:
        cp = dataclasses.replace(cp, needs_layout_passes=False)
    return pl.kernel(
        _sc_body,
        out_type=jax.ShapeDtypeStruct((S, D), jnp.float32),
        mesh=mesh,
        scratch_types=[
            pltpu.VMEM((C, D), jnp.float32),
            pltpu.VMEM((C, D), jnp.float32),
            pltpu.VMEM((C + L,), jnp.int32),
            pltpu.VMEM((C + L,), jnp.int32),
            pltpu.VMEM((C // HC, HC), jnp.int32),
            pltpu.VMEM((C // HC, HC), jnp.int32),
            pltpu.VMEM((SEG_PER,), jnp.float32),
            pltpu.VMEM((NB + L,), jnp.int32),
            pltpu.VMEM_SHARED((ACC_ROWS, D), jnp.float32),
            pltpu.SemaphoreType.DMA,
            pltpu.SemaphoreType.DMA,
        ],
        compiler_params=cp,
        interpret=interpret,
    )


@jax.jit
def kernel(X, cluster_assignment):
    ids = cluster_assignment.astype(jnp.int32)
    queries = jnp.arange(NW + 1, dtype=jnp.int32) * SEG_PER
    bounds = jnp.searchsorted(ids, queries, side="left",
                              method="compare_all").astype(jnp.int32)
    bounds = jnp.concatenate(
        [bounds, jnp.zeros((NB - NW - 1,), jnp.int32)])
    return _make_sc_kernel()(X, bounds, ids)


# speedup vs baseline: 1.3362x; 1.0002x over previous
import dataclasses

import jax
import jax.numpy as jnp
from jax import lax
from jax.experimental import pallas as pl
from jax.experimental.pallas import tpu as pltpu
from jax.experimental.pallas import tpu_sc as plsc

N = 320000
D = 128
S = 10000
L = 16
NW = 32
SEG_PER = 320
S_PAD = NW * SEG_PER
C = 256
HC = 128
NB = 48
SC_SEGS = 16 * SEG_PER
ACC_ROWS = SC_SEGS + 128
ZROWS = ACC_ROWS // 16
DUMP = SC_SEGS
EBLOCKS = ((0, 128), (128, 128), (256, 64))
ZBLOCKS = ((0, 128), (128, 128), (256, 72))


def _sc_body(x_hbm, bounds_hbm, ids_hbm, out_hbm,
             xbuf0, xbuf1, idb0, idb1, ix0, ix1, cnt_ref, bounds_vmem,
             acc_sh, sem0, sem1):
    core = lax.axis_index("c")
    sub = lax.axis_index("s")
    w = core * 16 + sub
    s_lo = w * SEG_PER
    sc_base = core * SC_SEGS
    iota = lax.iota(jnp.int32, L)
    ones = jnp.ones((L,), jnp.float32)

    pltpu.sync_copy(bounds_hbm, bounds_vmem.at[pl.ds(0, NB)])
    bv = bounds_vmem[pl.ds(w, L)]
    r_lo = bv[0]
    r_hi = bv[1]

    @pl.loop(0, SEG_PER, step=L)
    def _(k):
        cnt_ref[pl.ds(k, L)] = jnp.zeros((L,), jnp.float32)

    @pl.loop(0, HC)
    def _(r):
        for j in range(D // L):
            xbuf0[r, pl.ds(j * L, L)] = jnp.zeros((L,), jnp.float32)

    for off, sz in ZBLOCKS:
        pltpu.sync_copy(xbuf0.at[pl.ds(0, sz)],
                        acc_sh.at[pl.ds(sub * ZROWS + off, sz)])
    plsc.subcore_barrier()

    base = (r_lo // 16) * 16
    nchunks = (r_hi - base + C - 1) // C

    def chunk_start(cix):
        return jnp.minimum(base + cix * C, N - C)

    def issue(cix, xb, ib, sem):
        start = chunk_start(cix)
        pltpu.async_copy(x_hbm.at[pl.ds(start, C)], xb, sem)
        pltpu.async_copy(ids_hbm.at[pl.ds(start, C)], ib.at[pl.ds(0, C)], sem)

    def wait(xb, ib, sem):
        start = chunk_start(0)
        pltpu.make_async_copy(x_hbm.at[pl.ds(start, C)], xb, sem).wait()
        pltpu.make_async_copy(
            ids_hbm.at[pl.ds(start, C)], ib.at[pl.ds(0, C)], sem).wait()

    def process(cix, xb, ib, ix):
        start = chunk_start(cix)
        lo = jnp.maximum(r_lo, base + cix * C)
        hi = jnp.minimum(base + (cix + 1) * C, r_hi)

        for h in range(C // HC):
            for o in range(0, HC, L):
                iv = ib[pl.ds(h * HC + o, L)]
                rowv = start + h * HC + o + iota
                m = (rowv >= lo) & (rowv < hi)
                ix[h, pl.ds(o, L)] = jnp.where(m, iv - sc_base, DUMP)
                plsc.addupdate_scatter(cnt_ref, [iv - s_lo], ones, mask=m)

        for h in range(C // HC):
            pltpu.sync_copy(xb.at[pl.ds(h * HC, HC)],
                            acc_sh.at[ix.at[h]], add=True)

    @pl.when(nchunks > 0)
    def _():
        issue(0, xbuf0, idb0, sem0)

    def pair_body(p, _):
        c0 = 2 * p
        c1 = c0 + 1
        wait(xbuf0, idb0, sem0)

        @pl.when(c1 < nchunks)
        def _():
            issue(c1, xbuf1, idb1, sem1)

        process(c0, xbuf0, idb0, ix0)

        @pl.when(c1 < nchunks)
        def _():
            wait(xbuf1, idb1, sem1)

            @pl.when(c1 + 1 < nchunks)
            def _():
                issue(c1 + 1, xbuf0, idb0, sem0)

            process(c1, xbuf1, idb1, ix1)

        return 0

    lax.fori_loop(0, (nchunks + 1) // 2, pair_body, 0)

    plsc.subcore_barrier()

    for off, sz in EBLOCKS:
        pltpu.sync_copy(acc_sh.at[pl.ds(sub * SEG_PER + off, sz)],
                        xbuf0.at[pl.ds(0, sz)])

        @pl.loop(0, sz)
        def _(k):
            cv = plsc.load_gather(cnt_ref, [jnp.full((L,), off + k)])
            inv = 1.0 / jnp.maximum(cv, 1.0)
            vs = [xbuf0[k, pl.ds(j * L, L)] for j in range(D // L)]
            for j, v in enumerate(vs):
                xbuf0[k, pl.ds(j * L, L)] = v * inv

        @pl.when(w < NW - 1)
        def _():
            pltpu.sync_copy(xbuf0.at[pl.ds(0, sz)],
                            out_hbm.at[pl.ds(s_lo + off, sz)])

        if off == 0:
            @pl.when(w == NW - 1)
            def _():
                pltpu.sync_copy(xbuf0.at[pl.ds(0, S - (NW - 1) * SEG_PER)],
                                out_hbm.at[pl.ds(s_lo, S - (NW - 1) * SEG_PER)])


def _make_sc_kernel():
    mesh = plsc.VectorSubcoreMesh(
        core_axis_name="c", subcore_axis_name="s", num_cores=2, num_subcores=16
    )
    cp = pltpu.CompilerParams()
    if "needs_layout_passes" in pltpu.CompilerParams.__dataclass_fields__:
        cp = dataclasses.replace(cp, needs_layout_passes=False)
    return pl.kernel(
        _sc_body,
        out_type=jax.ShapeDtypeStruct((S, D), jnp.float32),
        mesh=mesh,
        scratch_types=[
            pltpu.VMEM((C, D), jnp.float32),
            pltpu.VMEM((C, D), jnp.float32),
            pltpu.VMEM((C + L,), jnp.int32),
            pltpu.VMEM((C + L,), jnp.int32),
            pltpu.VMEM((C // HC, HC), jnp.int32),
            pltpu.VMEM((C // HC, HC), jnp.int32),
            pltpu.VMEM((SEG_PER,), jnp.float32),
            pltpu.VMEM((NB + L,), jnp.int32),
            pltpu.VMEM_SHARED((ACC_ROWS, D), jnp.float32),
            pltpu.SemaphoreType.DMA,
            pltpu.SemaphoreType.DMA,
        ],
        compiler_params=cp,
    )


@jax.jit
def kernel(X, cluster_assignment):
    ids = cluster_assignment.astype(jnp.int32)
    queries = jnp.arange(NW + 1, dtype=jnp.int32) * SEG_PER
    bounds = jnp.searchsorted(ids, queries, side="left",
                              method="compare_all").astype(jnp.int32)
    bounds = jnp.concatenate(
        [bounds, jnp.zeros((NB - NW - 1,), jnp.int32)])
    return _make_sc_kernel()(X, bounds, ids)
